# baseline (speedup 1.0000x reference)
import jax
import jax.numpy as jnp
from jax import lax
from jax.experimental import pallas as pl
from jax.experimental.pallas import tpu as pltpu
from jax.experimental.pallas import tpu_sc as plsc

N_TOK = 10
D = 64
DP = 128
B = 1024
S = 200
V = 1000000
SEQ_G = S - N_TOK
HALF = SEQ_G // 2

NC = 2
NS = 16
NW = NC * NS
BPW = B // NW
CH = 2
NCH = BPW // CH

NBLK = V // DP
PER_W = NBLK // NW
EXTRA = NBLK - PER_W * NW
IN_BYTES = D * DP * 4
OUT_BYTES = DP * DP * 4

CGRP = 4


def _transpose_sc(tw, tail_pad, out, in_a, in_b, out_a, out_b, si_a, si_b, sw_a, sw_b):
    wid = lax.axis_index("s") * NC + lax.axis_index("c")
    base = wid * PER_W + jnp.minimum(wid, EXTRA)
    iota = lax.iota(jnp.int32, 16)
    rows = [iota + 16 * q for q in range(4)]

    def start_in(c, buf, sem):
        off = pl.multiple_of(c * DP, DP)
        return pltpu.async_copy(tw.at[:, pl.ds(off, DP)], buf, sem)

    def start_wr(c, buf, sem):
        off = pl.multiple_of(c * DP, DP)
        return pltpu.async_copy(buf, out.at[pl.ds(off, DP)], sem)

    def wait_in(buf, sem):
        pltpu.make_async_copy(tw.at[:, pl.ds(0, DP)], buf, sem).wait()

    def wait_wr(buf, sem):
        pltpu.make_async_copy(buf, out.at[pl.ds(0, DP)], sem).wait()

    def transpose(src, dst, width=DP):
        for v0 in range(0, width, CGRP):
            gs = []
            for v in range(v0, v0 + CGRP):
                col = jnp.full((16,), v, jnp.int32)
                for q in range(4):
                    gs.append((v, q, plsc.load_gather(src, [rows[q], col])))
            for v, q, g in gs:
                dst[v, pl.ds(16 * q, 16)] = g

    start_in(base, in_a, si_a)

    @pl.loop(0, PER_W // 2)
    def body(i):
        c0 = base + 2 * i
        start_in(c0 + 1, in_b, si_b)
        wait_in(in_a, si_a)

        @pl.when(i > 0)
        def _():
            wait_wr(out_a, sw_a)

        transpose(in_a, out_a)
        start_wr(c0, out_a, sw_a)
        start_in(jnp.where(2 * i + 2 < PER_W, c0 + 2, base), in_a, si_a)
        wait_in(in_b, si_b)

        @pl.when(i > 0)
        def _():
            wait_wr(out_b, sw_b)

        transpose(in_b, out_b)
        start_wr(c0 + 1, out_b, sw_b)

    wait_in(in_a, si_a)
    wait_wr(out_a, sw_a)
    wait_wr(out_b, sw_b)

    @pl.when(wid < EXTRA)
    def _extra():
        c = base + PER_W
        start_in(c, in_a, si_a).wait()
        transpose(in_a, out_a)
        start_wr(c, out_a, sw_a).wait()

    @pl.when(wid == NW - 1)
    def _tail():
        pltpu.async_copy(tail_pad, out_b, si_b).wait()
        pltpu.async_copy(out_b.at[pl.ds(0, V - NBLK * DP)],
                         out.at[pl.ds(NBLK * DP, V - NBLK * DP)], sw_b).wait()


def _soft_embed_sc(table, idx95, learned, out,
                   idx_v0, idx_v1, buf0, buf1, sem0, sem1):
    wid = lax.axis_index("s") * NC + lax.axis_index("c")
    idxs = (idx_v0, idx_v1)
    bufs = (buf0, buf1)
    sems = (sem0, sem1)

    for nb in range(2):
        for j in range(CH):
            pltpu.sync_copy(learned, bufs[nb].at[pl.ds(j * S, N_TOK), pl.ds(0, D)])

    def fetch(c, nb):
        b0 = wid * BPW + c * CH
        pltpu.sync_copy(idx95.at[pl.ds(b0 * 2, CH * 2)], idxs[nb])
        dmas = []
        for j in range(CH * 2):
            dst = bufs[nb].at[pl.ds((j // 2) * S + N_TOK + (j % 2) * HALF, HALF)]
            src = table.at[idxs[nb].at[j]]
            dmas.append(pltpu.async_copy(src, dst, sems[nb]))
        return dmas

    pending = fetch(0, 0)
    for c in range(NCH):
        nb = c % 2
        nxt = fetch(c + 1, 1 - nb) if c + 1 < NCH else None
        for d in pending:
            d.wait()
        b0 = wid * BPW + c * CH
        for j in range(CH):
            pltpu.sync_copy(bufs[nb].at[pl.ds(j * S, S), pl.ds(0, D)],
                            out.at[b0 + j])
        pending = nxt


def kernel(tokens, wte_weight, learned_embedding):
    idx95 = tokens[:, N_TOK:].reshape(B * 2, HALF)
    mesh = plsc.VectorSubcoreMesh(core_axis_name="c", subcore_axis_name="s")

    tr = pl.kernel(
        _transpose_sc,
        mesh=mesh,
        compiler_params=pltpu.CompilerParams(use_tc_tiling_on_sc=True,
                                             needs_layout_passes=False),
        out_type=jax.ShapeDtypeStruct((V, DP), jnp.float32),
        scratch_types=[
            pltpu.VMEM((D, DP), jnp.float32),
            pltpu.VMEM((D, DP), jnp.float32),
            pltpu.VMEM((DP, DP), jnp.float32),
            pltpu.VMEM((DP, DP), jnp.float32),
            pltpu.SemaphoreType.DMA,
            pltpu.SemaphoreType.DMA,
            pltpu.SemaphoreType.DMA,
            pltpu.SemaphoreType.DMA,
        ],
    )
    tail = V - NBLK * DP
    tail_pad = jnp.pad(wte_weight[NBLK * DP:, :], ((0, DP - tail), (0, DP - D)))
    table_pad = tr(wte_weight.T, tail_pad)

    emb = pl.kernel(
        _soft_embed_sc,
        mesh=mesh,
        compiler_params=pltpu.CompilerParams(use_tc_tiling_on_sc=False),
        out_type=jax.ShapeDtypeStruct((B, S, D), jnp.float32),
        scratch_types=[
            pltpu.VMEM((CH * 2, HALF), jnp.int32),
            pltpu.VMEM((CH * 2, HALF), jnp.int32),
            pltpu.VMEM((CH * S, DP), jnp.float32),
            pltpu.VMEM((CH * S, DP), jnp.float32),
            pltpu.SemaphoreType.DMA,
            pltpu.SemaphoreType.DMA,
        ],
    )
    return emb(table_pad, idx95, learned_embedding)


# speedup vs baseline: 3.4140x; 3.4140x over previous
import jax
import jax.numpy as jnp
from jax import lax
from jax.experimental import pallas as pl
from jax.experimental.pallas import tpu as pltpu
from jax.experimental.pallas import tpu_sc as plsc

N_TOK = 10
D = 64
DP = 128
B = 1024
S = 200
V = 1000000
SEQ_G = S - N_TOK
HALF = SEQ_G // 2

NC = 2
NS = 16
NW = NC * NS
BPW = B // NW
CH = 2
NCH = BPW // CH

NBLK = V // DP
PER_W = NBLK // NW
EXTRA = NBLK - PER_W * NW
IN_BYTES = D * DP * 4
OUT_BYTES = DP * DP * 4

CGRP = 4


def _transpose_sc(tw, tail_pad, out, in_a, in_b, out_a, out_b, si_a, si_b, sw_a, sw_b):
    wid = lax.axis_index("s") * NC + lax.axis_index("c")
    base = wid * PER_W + jnp.minimum(wid, EXTRA)
    iota = lax.iota(jnp.int32, 16)
    rows = [iota + 16 * q for q in range(4)]

    def start_in(c, buf, sem):
        off = pl.multiple_of(c * DP, DP)
        return pltpu.async_copy(tw.at[:, pl.ds(off, DP)], buf, sem)

    def start_wr(c, buf, sem):
        off = pl.multiple_of(c * DP, DP)
        return pltpu.async_copy(buf, out.at[pl.ds(off, DP)], sem)

    def wait_in(buf, sem):
        pltpu.make_async_copy(tw.at[:, pl.ds(0, DP)], buf, sem).wait()

    def wait_wr(buf, sem):
        pltpu.make_async_copy(buf, out.at[pl.ds(0, DP)], sem).wait()

    def transpose(src, dst, width=DP):
        for v0 in range(0, width, CGRP):
            gs = []
            for v in range(v0, v0 + CGRP):
                col = jnp.full((16,), v, jnp.int32)
                for q in range(4):
                    gs.append((v, q, plsc.load_gather(src, [rows[q], col])))
            for v, q, g in gs:
                dst[v, pl.ds(16 * q, 16)] = g

    start_in(base, in_a, si_a)

    @pl.loop(0, PER_W // 2)
    def body(i):
        c0 = base + 2 * i
        start_in(c0 + 1, in_b, si_b)
        wait_in(in_a, si_a)

        @pl.when(i > 0)
        def _():
            wait_wr(out_a, sw_a)

        transpose(in_a, out_a)
        start_wr(c0, out_a, sw_a)
        start_in(jnp.where(2 * i + 2 < PER_W, c0 + 2, base), in_a, si_a)
        wait_in(in_b, si_b)

        @pl.when(i > 0)
        def _():
            wait_wr(out_b, sw_b)

        transpose(in_b, out_b)
        start_wr(c0 + 1, out_b, sw_b)

    wait_in(in_a, si_a)
    wait_wr(out_a, sw_a)
    wait_wr(out_b, sw_b)

    @pl.when(wid < EXTRA)
    def _extra():
        c = base + PER_W
        start_in(c, in_a, si_a).wait()
        transpose(in_a, out_a)
        start_wr(c, out_a, sw_a).wait()

    @pl.when(wid == NW - 1)
    def _tail():
        pltpu.async_copy(tail_pad, out_b, si_b).wait()
        pltpu.async_copy(out_b.at[pl.ds(0, V - NBLK * DP)],
                         out.at[pl.ds(NBLK * DP, V - NBLK * DP)], sw_b).wait()


def _soft_embed_sc(table, idx95, learned, out,
                   idx_v0, idx_v1, buf0, buf1, sem0, sem1):
    wid = lax.axis_index("s") * NC + lax.axis_index("c")
    idxs = (idx_v0, idx_v1)
    bufs = (buf0, buf1)
    sems = (sem0, sem1)

    for nb in range(2):
        for j in range(CH):
            pltpu.sync_copy(learned, bufs[nb].at[pl.ds(j * S, N_TOK), pl.ds(0, D)])

    def fetch(c, nb):
        b0 = wid * BPW + c * CH
        pltpu.sync_copy(idx95.at[pl.ds(b0 * 2, CH * 2)], idxs[nb])
        dmas = []
        for j in range(CH * 2):
            dst = bufs[nb].at[pl.ds((j // 2) * S + N_TOK + (j % 2) * HALF, HALF)]
            src = table.at[idxs[nb].at[j]]
            dmas.append(pltpu.async_copy(src, dst, sems[nb]))
        return dmas

    pending = fetch(0, 0)
    for c in range(NCH):
        nb = c % 2
        nxt = fetch(c + 1, 1 - nb) if c + 1 < NCH else None
        for d in pending:
            d.wait()
        b0 = wid * BPW + c * CH
        for j in range(CH):
            pltpu.sync_copy(bufs[nb].at[pl.ds(j * S, S), pl.ds(0, D)],
                            out.at[b0 + j])
        pending = nxt


def kernel(tokens, wte_weight, learned_embedding):
    idx95 = tokens[:, N_TOK:].reshape(B * 2, HALF)
    mesh = plsc.VectorSubcoreMesh(core_axis_name="c", subcore_axis_name="s")

    tr = pl.kernel(
        _transpose_sc,
        mesh=mesh,
        compiler_params=pltpu.CompilerParams(use_tc_tiling_on_sc=True,
                                             needs_layout_passes=False),
        out_type=jax.ShapeDtypeStruct((V, DP), jnp.float32),
        scratch_types=[
            pltpu.VMEM((D, DP), jnp.float32),
            pltpu.VMEM((D, DP), jnp.float32),
            pltpu.VMEM((DP, DP), jnp.float32),
            pltpu.VMEM((DP, DP), jnp.float32),
            pltpu.SemaphoreType.DMA,
            pltpu.SemaphoreType.DMA,
            pltpu.SemaphoreType.DMA,
            pltpu.SemaphoreType.DMA,
        ],
    )
    del tr
    eye_pad = jnp.eye(D, DP, dtype=jnp.float32)
    table_pad = wte_weight @ eye_pad

    emb = pl.kernel(
        _soft_embed_sc,
        mesh=mesh,
        compiler_params=pltpu.CompilerParams(use_tc_tiling_on_sc=False),
        out_type=jax.ShapeDtypeStruct((B, S, D), jnp.float32),
        scratch_types=[
            pltpu.VMEM((CH * 2, HALF), jnp.int32),
            pltpu.VMEM((CH * 2, HALF), jnp.int32),
            pltpu.VMEM((CH * S, DP), jnp.float32),
            pltpu.VMEM((CH * S, DP), jnp.float32),
            pltpu.SemaphoreType.DMA,
            pltpu.SemaphoreType.DMA,
        ],
    )
    return emb(table_pad, idx95, learned_embedding)
